# baseline (speedup 1.0000x reference)
import functools

import jax
import jax.numpy as jnp
from jax import lax
from jax.experimental import pallas as pl
from jax.experimental.pallas import tpu as pltpu
from jax.experimental.pallas import tpu_sc as plsc

NC = 2
NS = 16
LANES = 16
KCH = 128
N_TRASH = 8


def _make_deg(n_cnt, e_pad):
  e_per_tile = e_pad // (NC * NS)
  mesh = plsc.VectorSubcoreMesh(core_axis_name="c", subcore_axis_name="s")

  @functools.partial(
      pl.kernel,
      out_type=jax.ShapeDtypeStruct((NC * NS, n_cnt), jnp.float32),
      mesh=mesh,
      scratch_types=[
          pltpu.VMEM((e_per_tile,), jnp.int32),
          pltpu.VMEM((n_cnt,), jnp.float32),
      ],
      compiler_params=pltpu.CompilerParams(needs_layout_passes=False),
  )
  def deg_kernel(dst_hbm, out_hbm, dst_v, cnt_v):
    c = lax.axis_index("c")
    s = lax.axis_index("s")
    w = c * NS + s
    zeros = jnp.zeros((LANES,), jnp.float32)

    def zbody(i, carry):
      cnt_v[pl.ds(i * LANES, LANES)] = zeros
      return carry

    lax.fori_loop(0, n_cnt // LANES, zbody, 0)
    pltpu.sync_copy(dst_hbm.at[pl.ds(w * e_per_tile, e_per_tile)], dst_v)
    ones = jnp.ones((LANES,), jnp.float32)

    def body(i, carry):
      idx = dst_v[pl.ds(i * LANES, LANES)]
      plsc.addupdate_scatter(cnt_v, [idx], ones)
      return carry

    lax.fori_loop(0, e_per_tile // LANES, body, 0)
    pltpu.sync_copy(cnt_v, out_hbm.at[w])

  return deg_kernel


def _make_agg(n_pad, dh, e_pad):
  rows_per_tile = n_pad // NS
  chunks_per_tile = e_pad // (NS * KCH)
  nbuf = 4
  n_groups = 2
  cpg = chunks_per_tile // n_groups
  assert chunks_per_tile == n_groups * cpg
  assert cpg % nbuf == 0 and cpg >= 2 * nbuf
  mesh = plsc.VectorSubcoreMesh(core_axis_name="c", subcore_axis_name="s")

  @functools.partial(
      pl.kernel,
      out_type=jax.ShapeDtypeStruct((NC, n_pad, dh), jnp.bfloat16),
      mesh=mesh,
      scratch_types=[
          pltpu.VMEM_SHARED((n_pad, dh), jnp.bfloat16),
          pltpu.VMEM((cpg, 2, KCH), jnp.int32),
          [pltpu.VMEM((KCH, dh), jnp.bfloat16)] * nbuf,
          [pltpu.SemaphoreType.DMA] * nbuf,
          [pltpu.SemaphoreType.DMA] * nbuf,
      ],
      compiler_params=pltpu.CompilerParams(
          needs_layout_passes=False, use_tc_tiling_on_sc=False),
  )
  def agg_kernel(hs_hbm, sd_hbm, out_hbm, acc, idx_sd, rows, sem_g, sem_s):
    c = lax.axis_index("c")
    s = lax.axis_index("s")
    base = s * rows_per_tile

    def gather(b, j):
      return pltpu.make_async_copy(hs_hbm.at[c].at[idx_sd.at[j, 0]], rows[b],
                                   sem_g[b])

    def scatter(b, j):
      return pltpu.make_async_copy(rows[b], acc.at[idx_sd.at[j, 1]], sem_s[b])

    pltpu.sync_copy(hs_hbm.at[c].at[pl.ds(base, rows_per_tile)],
                    acc.at[pl.ds(base, rows_per_tile)])
    plsc.subcore_barrier()

    for g in range(n_groups):
      pltpu.sync_copy(
          sd_hbm.at[pl.ds(s * chunks_per_tile + g * cpg, cpg)], idx_sd)
      for b in range(nbuf):
        gather(b, b).start()

      def body(t, carry):
        for b in range(nbuf):
          j = t * nbuf + b
          gather(b, j).wait()
          scatter(b, j).start(add=True)

          @pl.when(j + nbuf < cpg)
          def _():
            scatter(b, j).wait()
            gather(b, j + nbuf).start()
        return carry

      lax.fori_loop(0, cpg // nbuf, body, 0)
      for b in range(nbuf):
        scatter(b, 0).wait()

    plsc.subcore_barrier()
    pltpu.sync_copy(acc.at[pl.ds(base, rows_per_tile)],
                    out_hbm.at[c].at[pl.ds(base, rows_per_tile)])

  return agg_kernel


def _make_agg_es(n_pad, d, e_pad):
  rows_per_tile = n_pad // NS
  chunks_per_core = e_pad // (NC * KCH)
  cpt = chunks_per_core // NS
  nbuf = 4
  assert cpt % nbuf == 0
  zrows = 128
  assert rows_per_tile % zrows == 0
  mesh = plsc.VectorSubcoreMesh(core_axis_name="c", subcore_axis_name="s")

  @functools.partial(
      pl.kernel,
      out_type=jax.ShapeDtypeStruct((NC, n_pad, d), jnp.float32),
      mesh=mesh,
      scratch_types=[
          pltpu.VMEM_SHARED((n_pad, d), jnp.float32),
          pltpu.VMEM((cpt, 2, KCH), jnp.int32),
          pltpu.VMEM((zrows, d), jnp.float32),
          [pltpu.VMEM((KCH, d), jnp.float32)] * nbuf,
          [pltpu.SemaphoreType.DMA] * nbuf,
          [pltpu.SemaphoreType.DMA] * nbuf,
      ],
      compiler_params=pltpu.CompilerParams(
          needs_layout_passes=False, use_tc_tiling_on_sc=False),
  )
  def agg_kernel(hs_hbm, sd_hbm, out_hbm, acc, idx_sd, zbuf, rows, sem_g,
                 sem_s):
    c = lax.axis_index("c")
    s = lax.axis_index("s")
    base = s * rows_per_tile

    def gather(b, j):
      return pltpu.make_async_copy(hs_hbm.at[idx_sd.at[j, 0]], rows[b],
                                   sem_g[b])

    def scatter(b, j):
      return pltpu.make_async_copy(rows[b], acc.at[idx_sd.at[j, 1]], sem_s[b])

    @pl.when(c == 0)
    def _():
      pltpu.sync_copy(hs_hbm.at[pl.ds(base, rows_per_tile)],
                      acc.at[pl.ds(base, rows_per_tile)])

    @pl.when(c == 1)
    def _():
      zv = jnp.zeros((LANES,), jnp.float32)

      def zb(t, carry):
        i = t // (d // LANES)
        k = t % (d // LANES)
        zbuf[i, pl.ds(k * LANES, LANES)] = zv
        return carry

      lax.fori_loop(0, zrows * d // LANES, zb, 0)
      for r in range(rows_per_tile // zrows):
        pltpu.sync_copy(zbuf, acc.at[pl.ds(base + r * zrows, zrows)])

    plsc.subcore_barrier()
    pltpu.sync_copy(
        sd_hbm.at[pl.ds(c * chunks_per_core + s * cpt, cpt)], idx_sd)
    for b in range(nbuf):
      gather(b, b).start()

    def body(t, carry):
      for b in range(nbuf):
        j = t * nbuf + b
        gather(b, j).wait()
        scatter(b, j).start(add=True)

        @pl.when(j + nbuf < cpt)
        def _():
          scatter(b, j).wait()
          gather(b, j + nbuf).start()
      return carry

    lax.fori_loop(0, cpt // nbuf, body, 0)
    for b in range(nbuf):
      scatter(b, 0).wait()
    plsc.subcore_barrier()
    pltpu.sync_copy(acc.at[pl.ds(base, rows_per_tile)],
                    out_hbm.at[c].at[pl.ds(base, rows_per_tile)])

  return agg_kernel


def _dis_from_parts(degp_block):
  deg = jnp.sum(degp_block, axis=0) + 1.0
  return lax.rsqrt(deg)


def _make_mm1(n, n_pad, d_in, d_hid, bm):
  dh = d_hid // NC

  def body(x_ref, w_ref, degp_ref, o_ref):
    dis = _dis_from_parts(degp_ref[...])
    h = jnp.dot(x_ref[...], w_ref[...], preferred_element_type=jnp.float32)
    o_ref[0] = (h * dis[:, None]).astype(jnp.bfloat16)

  return pl.pallas_call(
      body,
      grid=(NC, n_pad // bm),
      in_specs=[
          pl.BlockSpec((bm, d_in), lambda c, i: (i, 0)),
          pl.BlockSpec((d_in, dh), lambda c, i: (0, c)),
          pl.BlockSpec((NC * NS, bm), lambda c, i: (0, i)),
      ],
      out_specs=pl.BlockSpec((1, bm, dh), lambda c, i: (c, i, 0)),
      out_shape=jax.ShapeDtypeStruct((NC, n_pad, dh), jnp.bfloat16),
  )


def _make_mm2(n_pad, d_hid, d_out, bm):
  dh_in = d_hid // NC

  def body(a_ref, degp_ref, b1_ref, w2_ref, o_ref):
    dis = _dis_from_parts(degp_ref[...])
    z = jnp.concatenate([a_ref[0], a_ref[1]], axis=-1).astype(jnp.float32)
    z = jnp.maximum(z * dis[:, None] + b1_ref[...], 0.0)
    h = jnp.dot(z, w2_ref[...], preferred_element_type=jnp.float32)
    o_ref[...] = h * dis[:, None]

  return pl.pallas_call(
      body,
      grid=(n_pad // bm,),
      in_specs=[
          pl.BlockSpec((NC, bm, dh_in), lambda i: (0, i, 0)),
          pl.BlockSpec((NC * NS, bm), lambda i: (0, i)),
          pl.BlockSpec((1, d_hid), lambda i: (0, 0)),
          pl.BlockSpec((d_hid, d_out), lambda i: (0, 0)),
      ],
      out_specs=pl.BlockSpec((bm, d_out), lambda i: (i, 0)),
      out_shape=jax.ShapeDtypeStruct((n_pad, d_out), jnp.float32),
  )


def _make_fin(n, n_pad, d_out, n_cls, bm):
  def body(a_ref, degp_ref, b2_ref, wc_ref, bc_ref, o_ref):
    dis = _dis_from_parts(degp_ref[...])
    z = a_ref[0] + a_ref[1]
    z = z * dis[:, None] + b2_ref[...]
    o_ref[...] = (
        jnp.dot(z, wc_ref[...], preferred_element_type=jnp.float32)
        + bc_ref[...])

  return pl.pallas_call(
      body,
      grid=(n_pad // bm,),
      in_specs=[
          pl.BlockSpec((NC, bm, d_out), lambda i: (0, i, 0)),
          pl.BlockSpec((NC * NS, bm), lambda i: (0, i)),
          pl.BlockSpec((1, d_out), lambda i: (0, 0)),
          pl.BlockSpec((d_out, n_cls), lambda i: (0, 0)),
          pl.BlockSpec((1, n_cls), lambda i: (0, 0)),
      ],
      out_specs=pl.BlockSpec((bm, n_cls), lambda i: (i, 0)),
      out_shape=jax.ShapeDtypeStruct((n, n_cls), jnp.float32),
  )


def kernel(x, edge_index, W1, b1, W2, b2, Wc, bc):
  n, d_in = x.shape
  d_hid = W1.shape[1]
  d_out = W2.shape[1]
  n_cls = Wc.shape[1]
  e = edge_index.shape[1]

  quantum = 2 * NS * KCH
  e_pad = ((e + quantum - 1) // quantum) * quantum
  pad = e_pad - e
  src = edge_index[0].astype(jnp.int32)
  dst = edge_index[1].astype(jnp.int32)
  bm = 1024
  n_pad = ((n + bm - 1) // bm) * bm
  src_p = jnp.concatenate([src, jnp.zeros((pad,), jnp.int32)])
  dst_p = jnp.concatenate(
      [dst, n + (jnp.arange(pad, dtype=jnp.int32) % (n_pad - n))])
  sd2 = jnp.stack([src_p.reshape(-1, KCH), dst_p.reshape(-1, KCH)], axis=1)
  degp = _make_deg(n_pad, e_pad)(dst_p)
  hs1 = _make_mm1(n, n_pad, d_in, d_hid, bm)(x, W1, degp)
  acc1 = _make_agg(n_pad, d_hid // NC, e_pad)(hs1, sd2)
  hs2 = _make_mm2(n_pad, d_hid, d_out, bm)(acc1, degp, b1.reshape(1, -1), W2)
  acc2 = _make_agg_es(n_pad, d_out, e_pad)(hs2, sd2)
  logits = _make_fin(n, n_pad, d_out, n_cls, bm)(
      acc2, degp, b2.reshape(1, -1), Wc, bc.reshape(1, -1))
  return logits


# speedup vs baseline: 1.0039x; 1.0039x over previous
import functools

import jax
import jax.numpy as jnp
from jax import lax
from jax.experimental import pallas as pl
from jax.experimental.pallas import tpu as pltpu
from jax.experimental.pallas import tpu_sc as plsc

NC = 2
NS = 16
LANES = 16
KCH = 128
N_TRASH = 8


def _make_deg(n_cnt, e_pad):
  e_per_tile = e_pad // (NC * NS)
  mesh = plsc.VectorSubcoreMesh(core_axis_name="c", subcore_axis_name="s")

  @functools.partial(
      pl.kernel,
      out_type=jax.ShapeDtypeStruct((NC * NS, n_cnt), jnp.float32),
      mesh=mesh,
      scratch_types=[
          pltpu.VMEM((e_per_tile,), jnp.int32),
          pltpu.VMEM((n_cnt,), jnp.float32),
      ],
      compiler_params=pltpu.CompilerParams(needs_layout_passes=False),
  )
  def deg_kernel(dst_hbm, out_hbm, dst_v, cnt_v):
    c = lax.axis_index("c")
    s = lax.axis_index("s")
    w = c * NS + s
    zeros = jnp.zeros((LANES,), jnp.float32)

    def zbody(i, carry):
      cnt_v[pl.ds(i * LANES, LANES)] = zeros
      return carry

    lax.fori_loop(0, n_cnt // LANES, zbody, 0)
    pltpu.sync_copy(dst_hbm.at[pl.ds(w * e_per_tile, e_per_tile)], dst_v)
    ones = jnp.ones((LANES,), jnp.float32)

    def body(i, carry):
      idx = dst_v[pl.ds(i * LANES, LANES)]
      plsc.addupdate_scatter(cnt_v, [idx], ones)
      return carry

    lax.fori_loop(0, e_per_tile // LANES, body, 0)
    pltpu.sync_copy(cnt_v, out_hbm.at[w])

  return deg_kernel


def _make_agg(n_pad, dh, e_pad):
  rows_per_tile = n_pad // NS
  chunks_per_tile = e_pad // (NS * KCH)
  nbuf = 4
  n_groups = 2
  cpg = chunks_per_tile // n_groups
  assert chunks_per_tile == n_groups * cpg
  assert cpg % nbuf == 0 and cpg >= 2 * nbuf
  mesh = plsc.VectorSubcoreMesh(core_axis_name="c", subcore_axis_name="s")

  @functools.partial(
      pl.kernel,
      out_type=jax.ShapeDtypeStruct((NC, n_pad, dh), jnp.bfloat16),
      mesh=mesh,
      scratch_types=[
          pltpu.VMEM_SHARED((n_pad, dh), jnp.bfloat16),
          pltpu.VMEM((cpg, 2, KCH), jnp.int32),
          [pltpu.VMEM((KCH, dh), jnp.bfloat16)] * nbuf,
          [pltpu.SemaphoreType.DMA] * nbuf,
          [pltpu.SemaphoreType.DMA] * nbuf,
      ],
      compiler_params=pltpu.CompilerParams(
          needs_layout_passes=False, use_tc_tiling_on_sc=False),
  )
  def agg_kernel(hs_hbm, sd_hbm, out_hbm, acc, idx_sd, rows, sem_g, sem_s):
    c = lax.axis_index("c")
    s = lax.axis_index("s")
    base = s * rows_per_tile

    def gather(b, j):
      return pltpu.make_async_copy(hs_hbm.at[c].at[idx_sd.at[j, 0]], rows[b],
                                   sem_g[b])

    def scatter(b, j):
      return pltpu.make_async_copy(rows[b], acc.at[idx_sd.at[j, 1]], sem_s[b])

    pltpu.sync_copy(hs_hbm.at[c].at[pl.ds(base, rows_per_tile)],
                    acc.at[pl.ds(base, rows_per_tile)])
    plsc.subcore_barrier()

    for g in range(n_groups):
      pltpu.sync_copy(
          sd_hbm.at[pl.ds(s * chunks_per_tile + g * cpg, cpg)], idx_sd)
      for b in range(nbuf):
        gather(b, b).start()

      def body(t, carry):
        for b in range(nbuf):
          j = t * nbuf + b
          gather(b, j).wait()
          scatter(b, j).start(add=True)

          @pl.when(j + nbuf < cpg)
          def _():
            scatter(b, j).wait()
            gather(b, j + nbuf).start()
        return carry

      lax.fori_loop(0, cpg // nbuf, body, 0)
      for b in range(nbuf):
        scatter(b, 0).wait()

    plsc.subcore_barrier()
    pltpu.sync_copy(acc.at[pl.ds(base, rows_per_tile)],
                    out_hbm.at[c].at[pl.ds(base, rows_per_tile)])

  return agg_kernel


def _make_agg_es(n_pad, d, e_pad):
  rows_per_tile = n_pad // NS
  chunks_per_core = e_pad // (NC * KCH)
  cpt = chunks_per_core // NS
  nbuf = 4
  assert cpt % nbuf == 0
  zrows = 128
  assert rows_per_tile % zrows == 0
  mesh = plsc.VectorSubcoreMesh(core_axis_name="c", subcore_axis_name="s")

  @functools.partial(
      pl.kernel,
      out_type=jax.ShapeDtypeStruct((NC, n_pad, d), jnp.float32),
      mesh=mesh,
      scratch_types=[
          pltpu.VMEM_SHARED((n_pad, d), jnp.float32),
          pltpu.VMEM((cpt, 2, KCH), jnp.int32),
          pltpu.VMEM((zrows, d), jnp.float32),
          [pltpu.VMEM((KCH, d), jnp.float32)] * nbuf,
          [pltpu.SemaphoreType.DMA] * nbuf,
          [pltpu.SemaphoreType.DMA] * nbuf,
      ],
      compiler_params=pltpu.CompilerParams(
          needs_layout_passes=False, use_tc_tiling_on_sc=False),
  )
  def agg_kernel(hs_hbm, sd_hbm, out_hbm, acc, idx_sd, zbuf, rows, sem_g,
                 sem_s):
    c = lax.axis_index("c")
    s = lax.axis_index("s")
    base = s * rows_per_tile

    def gather(b, j):
      return pltpu.make_async_copy(hs_hbm.at[idx_sd.at[j, 0]], rows[b],
                                   sem_g[b])

    def scatter(b, j):
      return pltpu.make_async_copy(rows[b], acc.at[idx_sd.at[j, 1]], sem_s[b])

    @pl.when(c == 0)
    def _():
      pltpu.sync_copy(hs_hbm.at[pl.ds(base, rows_per_tile)],
                      acc.at[pl.ds(base, rows_per_tile)])

    @pl.when(c == 1)
    def _():
      zv = jnp.zeros((LANES,), jnp.float32)

      def zb(i, carry):
        for k in range(d // LANES):
          zbuf[i, pl.ds(k * LANES, LANES)] = zv
        return carry

      lax.fori_loop(0, zrows, zb, 0)
      for r in range(rows_per_tile // zrows):
        pltpu.sync_copy(zbuf, acc.at[pl.ds(base + r * zrows, zrows)])

    plsc.subcore_barrier()
    pltpu.sync_copy(
        sd_hbm.at[pl.ds(c * chunks_per_core + s * cpt, cpt)], idx_sd)
    for b in range(nbuf):
      gather(b, b).start()

    def body(t, carry):
      for b in range(nbuf):
        j = t * nbuf + b
        gather(b, j).wait()
        scatter(b, j).start(add=True)

        @pl.when(j + nbuf < cpt)
        def _():
          scatter(b, j).wait()
          gather(b, j + nbuf).start()
      return carry

    lax.fori_loop(0, cpt // nbuf, body, 0)
    for b in range(nbuf):
      scatter(b, 0).wait()
    plsc.subcore_barrier()
    pltpu.sync_copy(acc.at[pl.ds(base, rows_per_tile)],
                    out_hbm.at[c].at[pl.ds(base, rows_per_tile)])

  return agg_kernel


def _dis_from_parts(degp_block):
  deg = jnp.sum(degp_block, axis=0) + 1.0
  return lax.rsqrt(deg)


def _make_mm1(n, n_pad, d_in, d_hid, bm):
  dh = d_hid // NC

  def body(x_ref, w_ref, degp_ref, o_ref):
    dis = _dis_from_parts(degp_ref[...])
    h = jnp.dot(x_ref[...], w_ref[...], preferred_element_type=jnp.float32)
    o_ref[0] = (h * dis[:, None]).astype(jnp.bfloat16)

  return pl.pallas_call(
      body,
      grid=(NC, n_pad // bm),
      in_specs=[
          pl.BlockSpec((bm, d_in), lambda c, i: (i, 0)),
          pl.BlockSpec((d_in, dh), lambda c, i: (0, c)),
          pl.BlockSpec((NC * NS, bm), lambda c, i: (0, i)),
      ],
      out_specs=pl.BlockSpec((1, bm, dh), lambda c, i: (c, i, 0)),
      out_shape=jax.ShapeDtypeStruct((NC, n_pad, dh), jnp.bfloat16),
  )


def _make_mm2(n_pad, d_hid, d_out, bm):
  dh_in = d_hid // NC

  def body(a_ref, degp_ref, b1_ref, w2_ref, o_ref):
    dis = _dis_from_parts(degp_ref[...])
    z = jnp.concatenate([a_ref[0], a_ref[1]], axis=-1).astype(jnp.float32)
    z = jnp.maximum(z * dis[:, None] + b1_ref[...], 0.0)
    h = jnp.dot(z, w2_ref[...], preferred_element_type=jnp.float32)
    o_ref[...] = h * dis[:, None]

  return pl.pallas_call(
      body,
      grid=(n_pad // bm,),
      in_specs=[
          pl.BlockSpec((NC, bm, dh_in), lambda i: (0, i, 0)),
          pl.BlockSpec((NC * NS, bm), lambda i: (0, i)),
          pl.BlockSpec((1, d_hid), lambda i: (0, 0)),
          pl.BlockSpec((d_hid, d_out), lambda i: (0, 0)),
      ],
      out_specs=pl.BlockSpec((bm, d_out), lambda i: (i, 0)),
      out_shape=jax.ShapeDtypeStruct((n_pad, d_out), jnp.float32),
  )


def _make_fin(n, n_pad, d_out, n_cls, bm):
  def body(a_ref, degp_ref, b2_ref, wc_ref, bc_ref, o_ref):
    dis = _dis_from_parts(degp_ref[...])
    z = a_ref[0] + a_ref[1]
    z = z * dis[:, None] + b2_ref[...]
    o_ref[...] = (
        jnp.dot(z, wc_ref[...], preferred_element_type=jnp.float32)
        + bc_ref[...])

  return pl.pallas_call(
      body,
      grid=(n_pad // bm,),
      in_specs=[
          pl.BlockSpec((NC, bm, d_out), lambda i: (0, i, 0)),
          pl.BlockSpec((NC * NS, bm), lambda i: (0, i)),
          pl.BlockSpec((1, d_out), lambda i: (0, 0)),
          pl.BlockSpec((d_out, n_cls), lambda i: (0, 0)),
          pl.BlockSpec((1, n_cls), lambda i: (0, 0)),
      ],
      out_specs=pl.BlockSpec((bm, n_cls), lambda i: (i, 0)),
      out_shape=jax.ShapeDtypeStruct((n, n_cls), jnp.float32),
  )


def kernel(x, edge_index, W1, b1, W2, b2, Wc, bc):
  n, d_in = x.shape
  d_hid = W1.shape[1]
  d_out = W2.shape[1]
  n_cls = Wc.shape[1]
  e = edge_index.shape[1]

  quantum = 2 * NS * KCH
  e_pad = ((e + quantum - 1) // quantum) * quantum
  pad = e_pad - e
  src = edge_index[0].astype(jnp.int32)
  dst = edge_index[1].astype(jnp.int32)
  bm = 1024
  n_pad = ((n + bm - 1) // bm) * bm
  src_p = jnp.concatenate([src, jnp.zeros((pad,), jnp.int32)])
  dst_p = jnp.concatenate(
      [dst, n + (jnp.arange(pad, dtype=jnp.int32) % (n_pad - n))])
  sd2 = jnp.stack([src_p.reshape(-1, KCH), dst_p.reshape(-1, KCH)], axis=1)
  degp = _make_deg(n_pad, e_pad)(dst_p)
  hs1 = _make_mm1(n, n_pad, d_in, d_hid, bm)(x, W1, degp)
  acc1 = _make_agg(n_pad, d_hid // NC, e_pad)(hs1, sd2)
  hs2 = _make_mm2(n_pad, d_hid, d_out, bm)(acc1, degp, b1.reshape(1, -1), W2)
  acc2 = _make_agg_es(n_pad, d_out, e_pad)(hs2, sd2)
  logits = _make_fin(n, n_pad, d_out, n_cls, bm)(
      acc2, degp, b2.reshape(1, -1), Wc, bc.reshape(1, -1))
  return logits


# speedup vs baseline: 2.0006x; 1.9927x over previous
import functools

import jax
import jax.numpy as jnp
from jax import lax
from jax.experimental import pallas as pl
from jax.experimental.pallas import tpu as pltpu
from jax.experimental.pallas import tpu_sc as plsc

NC = 2
NS = 16
LANES = 16
KCH = 128
N_TRASH = 8


def _make_deg(n_cnt, e_pad):
  e_per_tile = e_pad // (NC * NS)
  mesh = plsc.VectorSubcoreMesh(core_axis_name="c", subcore_axis_name="s")

  @functools.partial(
      pl.kernel,
      out_type=jax.ShapeDtypeStruct((NC * NS, n_cnt), jnp.float32),
      mesh=mesh,
      scratch_types=[
          pltpu.VMEM((e_per_tile,), jnp.int32),
          pltpu.VMEM((n_cnt,), jnp.float32),
      ],
      compiler_params=pltpu.CompilerParams(needs_layout_passes=False),
  )
  def deg_kernel(dst_hbm, out_hbm, dst_v, cnt_v):
    c = lax.axis_index("c")
    s = lax.axis_index("s")
    w = c * NS + s
    zeros = jnp.zeros((LANES,), jnp.float32)

    def zbody(i, carry):
      cnt_v[pl.ds(i * LANES, LANES)] = zeros
      return carry

    lax.fori_loop(0, n_cnt // LANES, zbody, 0)
    pltpu.sync_copy(dst_hbm.at[pl.ds(w * e_per_tile, e_per_tile)], dst_v)
    ones = jnp.ones((LANES,), jnp.float32)

    def body(i, carry):
      idx = dst_v[pl.ds(i * LANES, LANES)]
      plsc.addupdate_scatter(cnt_v, [idx], ones)
      return carry

    lax.fori_loop(0, e_per_tile // LANES, body, 0)
    pltpu.sync_copy(cnt_v, out_hbm.at[w])

  return deg_kernel


def _make_agg(n_pad, dh, e_pad):
  rows_per_tile = n_pad // NS
  chunks_per_tile = e_pad // (NS * KCH)
  nbuf = 4
  n_groups = 2
  cpg = chunks_per_tile // n_groups
  assert chunks_per_tile == n_groups * cpg
  assert cpg % nbuf == 0 and cpg >= 2 * nbuf
  mesh = plsc.VectorSubcoreMesh(core_axis_name="c", subcore_axis_name="s")

  @functools.partial(
      pl.kernel,
      out_type=jax.ShapeDtypeStruct((NC, n_pad, dh), jnp.bfloat16),
      mesh=mesh,
      scratch_types=[
          pltpu.VMEM_SHARED((n_pad, dh), jnp.bfloat16),
          pltpu.VMEM((cpg, 2, KCH), jnp.int32),
          [pltpu.VMEM((KCH, dh), jnp.bfloat16)] * nbuf,
          [pltpu.SemaphoreType.DMA] * nbuf,
          [pltpu.SemaphoreType.DMA] * nbuf,
      ],
      compiler_params=pltpu.CompilerParams(
          needs_layout_passes=False, use_tc_tiling_on_sc=False),
  )
  def agg_kernel(hs_hbm, sd_hbm, out_hbm, acc, idx_sd, rows, sem_g, sem_s):
    c = lax.axis_index("c")
    s = lax.axis_index("s")
    base = s * rows_per_tile

    def gather(b, j):
      return pltpu.make_async_copy(hs_hbm.at[c].at[idx_sd.at[j, 0]], rows[b],
                                   sem_g[b])

    def scatter(b, j):
      return pltpu.make_async_copy(rows[b], acc.at[idx_sd.at[j, 1]], sem_s[b])

    pltpu.sync_copy(hs_hbm.at[c].at[pl.ds(base, rows_per_tile)],
                    acc.at[pl.ds(base, rows_per_tile)])
    plsc.subcore_barrier()

    for g in range(n_groups):
      pltpu.sync_copy(
          sd_hbm.at[pl.ds(s * chunks_per_tile + g * cpg, cpg)], idx_sd)
      for b in range(nbuf):
        gather(b, b).start()

      def body(t, carry):
        for b in range(nbuf):
          j = t * nbuf + b
          gather(b, j).wait()
          scatter(b, j).start(add=True)

          @pl.when(j + nbuf < cpg)
          def _():
            scatter(b, j).wait()
            gather(b, j + nbuf).start()
        return carry

      lax.fori_loop(0, cpg // nbuf, body, 0)
      for b in range(nbuf):
        scatter(b, 0).wait()

    plsc.subcore_barrier()
    pltpu.sync_copy(acc.at[pl.ds(base, rows_per_tile)],
                    out_hbm.at[c].at[pl.ds(base, rows_per_tile)])

  return agg_kernel


def _make_agg_es(n_pad, d, e_pad):
  rows_per_tile = n_pad // NS
  chunks_per_core = e_pad // (NC * KCH)
  cpt = chunks_per_core // NS
  nbuf = 4
  assert cpt % nbuf == 0
  zrows = 128
  assert rows_per_tile % zrows == 0
  mesh = plsc.VectorSubcoreMesh(core_axis_name="c", subcore_axis_name="s")

  @functools.partial(
      pl.kernel,
      out_type=jax.ShapeDtypeStruct((NC, n_pad, d), jnp.float32),
      mesh=mesh,
      scratch_types=[
          pltpu.VMEM_SHARED((n_pad, d), jnp.float32),
          pltpu.VMEM((cpt, 2, KCH), jnp.int32),
          pltpu.VMEM((zrows, d), jnp.float32),
          [pltpu.VMEM((KCH, d), jnp.float32)] * nbuf,
          [pltpu.SemaphoreType.DMA] * nbuf,
          [pltpu.SemaphoreType.DMA] * nbuf,
      ],
      compiler_params=pltpu.CompilerParams(
          needs_layout_passes=False, use_tc_tiling_on_sc=False),
  )
  def agg_kernel(hs_hbm, sd_hbm, out_hbm, acc, idx_sd, zbuf, rows, sem_g,
                 sem_s):
    c = lax.axis_index("c")
    s = lax.axis_index("s")
    base = s * rows_per_tile

    def gather(b, j):
      return pltpu.make_async_copy(hs_hbm.at[idx_sd.at[j, 0]], rows[b],
                                   sem_g[b])

    def scatter(b, j):
      return pltpu.make_async_copy(rows[b], acc.at[idx_sd.at[j, 1]], sem_s[b])

    @pl.when(c == 0)
    def _():
      pltpu.sync_copy(hs_hbm.at[pl.ds(base, rows_per_tile)],
                      acc.at[pl.ds(base, rows_per_tile)])

    @pl.when(c == 1)
    def _():
      zv = jnp.zeros((LANES,), jnp.float32)

      def zb(i, carry):
        for k in range(d // LANES):
          zbuf[i, pl.ds(k * LANES, LANES)] = zv
        return carry

      lax.fori_loop(0, zrows, zb, 0)
      for r in range(rows_per_tile // zrows):
        pltpu.sync_copy(zbuf, acc.at[pl.ds(base + r * zrows, zrows)])

    plsc.subcore_barrier()
    pltpu.sync_copy(
        sd_hbm.at[pl.ds(c * chunks_per_core + s * cpt, cpt)], idx_sd)
    for b in range(nbuf):
      gather(b, b).start()

    def body(t, carry):
      for b in range(nbuf):
        j = t * nbuf + b
        gather(b, j).wait()
        scatter(b, j).start(add=True)

        @pl.when(j + nbuf < cpt)
        def _():
          scatter(b, j).wait()
          gather(b, j + nbuf).start()
      return carry

    lax.fori_loop(0, cpt // nbuf, body, 0)
    for b in range(nbuf):
      scatter(b, 0).wait()
    plsc.subcore_barrier()
    pltpu.sync_copy(acc.at[pl.ds(base, rows_per_tile)],
                    out_hbm.at[c].at[pl.ds(base, rows_per_tile)])

  return agg_kernel


def _dis_from_parts(degp_block):
  deg = jnp.sum(degp_block, axis=0) + 1.0
  return lax.rsqrt(deg)


def _make_mm1(n, n_pad, d_in, d_hid, bm):
  dh = d_hid // NC

  def body(x_ref, w_ref, degp_ref, o_ref):
    dis = _dis_from_parts(degp_ref[...])
    h = jnp.dot(x_ref[...], w_ref[...], preferred_element_type=jnp.float32)
    o_ref[0] = (h * dis[:, None]).astype(jnp.bfloat16)

  return pl.pallas_call(
      body,
      grid=(NC, n_pad // bm),
      in_specs=[
          pl.BlockSpec((bm, d_in), lambda c, i: (i, 0)),
          pl.BlockSpec((d_in, dh), lambda c, i: (0, c)),
          pl.BlockSpec((NC * NS, bm), lambda c, i: (0, i)),
      ],
      out_specs=pl.BlockSpec((1, bm, dh), lambda c, i: (c, i, 0)),
      out_shape=jax.ShapeDtypeStruct((NC, n_pad, dh), jnp.bfloat16),
  )


def _make_mm2(n_pad, d_hid, d_out, bm):
  dh_in = d_hid // NC

  def body(a_ref, degp_ref, b1_ref, w2_ref, o_ref):
    dis = _dis_from_parts(degp_ref[...])
    z = jnp.concatenate([a_ref[0], a_ref[1]], axis=-1).astype(jnp.float32)
    z = jnp.maximum(z * dis[:, None] + b1_ref[...], 0.0)
    h = jnp.dot(z, w2_ref[...], preferred_element_type=jnp.float32)
    o_ref[...] = h * dis[:, None]

  return pl.pallas_call(
      body,
      grid=(n_pad // bm,),
      in_specs=[
          pl.BlockSpec((NC, bm, dh_in), lambda i: (0, i, 0)),
          pl.BlockSpec((NC * NS, bm), lambda i: (0, i)),
          pl.BlockSpec((1, d_hid), lambda i: (0, 0)),
          pl.BlockSpec((d_hid, d_out), lambda i: (0, 0)),
      ],
      out_specs=pl.BlockSpec((bm, d_out), lambda i: (i, 0)),
      out_shape=jax.ShapeDtypeStruct((n_pad, d_out), jnp.float32),
  )


def _make_fin(n, n_pad, d_out, n_cls, bm):
  def body(a_ref, degp_ref, b2_ref, wc_ref, bc_ref, o_ref):
    dis = _dis_from_parts(degp_ref[...])
    z = a_ref[0] + a_ref[1]
    z = z * dis[:, None] + b2_ref[...]
    o_ref[...] = (
        jnp.dot(z, wc_ref[...], preferred_element_type=jnp.float32)
        + bc_ref[...])

  return pl.pallas_call(
      body,
      grid=(n_pad // bm,),
      in_specs=[
          pl.BlockSpec((NC, bm, d_out), lambda i: (0, i, 0)),
          pl.BlockSpec((NC * NS, bm), lambda i: (0, i)),
          pl.BlockSpec((1, d_out), lambda i: (0, 0)),
          pl.BlockSpec((d_out, n_cls), lambda i: (0, 0)),
          pl.BlockSpec((1, n_cls), lambda i: (0, 0)),
      ],
      out_specs=pl.BlockSpec((bm, n_cls), lambda i: (i, 0)),
      out_shape=jax.ShapeDtypeStruct((n, n_cls), jnp.float32),
  )


def kernel(x, edge_index, W1, b1, W2, b2, Wc, bc):
  n, d_in = x.shape
  d_hid = W1.shape[1]
  d_out = W2.shape[1]
  n_cls = Wc.shape[1]
  e = edge_index.shape[1]

  quantum = 2 * NS * KCH
  e_pad = ((e + quantum - 1) // quantum) * quantum
  pad = e_pad - e
  src = edge_index[0].astype(jnp.int32)
  dst = edge_index[1].astype(jnp.int32)
  bm = 1024
  n_pad = ((n + bm - 1) // bm) * bm
  src_p = jnp.concatenate([src, jnp.arange(pad, dtype=jnp.int32) % n])
  dst_p = jnp.concatenate(
      [dst, n + (jnp.arange(pad, dtype=jnp.int32) % (n_pad - n))])
  sd2 = jnp.stack([src_p.reshape(-1, KCH), dst_p.reshape(-1, KCH)], axis=1)
  degp = _make_deg(n_pad, e_pad)(dst_p)
  hs1 = _make_mm1(n, n_pad, d_in, d_hid, bm)(x, W1, degp)
  acc1 = _make_agg(n_pad, d_hid // NC, e_pad)(hs1, sd2)
  hs2 = _make_mm2(n_pad, d_hid, d_out, bm)(acc1, degp, b1.reshape(1, -1), W2)
  acc2 = _make_agg_es(n_pad, d_out, e_pad)(hs2, sd2)
  logits = _make_fin(n, n_pad, d_out, n_cls, bm)(
      acc2, degp, b2.reshape(1, -1), Wc, bc.reshape(1, -1))
  return logits


# speedup vs baseline: 2.0011x; 1.0002x over previous
import functools

import jax
import jax.numpy as jnp
from jax import lax
from jax.experimental import pallas as pl
from jax.experimental.pallas import tpu as pltpu
from jax.experimental.pallas import tpu_sc as plsc

NC = 2
NS = 16
LANES = 16
KCH = 128
N_TRASH = 8


def _make_deg(n_cnt, e_pad):
  e_per_tile = e_pad // (NC * NS)
  mesh = plsc.VectorSubcoreMesh(core_axis_name="c", subcore_axis_name="s")

  @functools.partial(
      pl.kernel,
      out_type=jax.ShapeDtypeStruct((NC * NS, n_cnt), jnp.float32),
      mesh=mesh,
      scratch_types=[
          pltpu.VMEM((e_per_tile,), jnp.int32),
          pltpu.VMEM((n_cnt,), jnp.float32),
      ],
      compiler_params=pltpu.CompilerParams(needs_layout_passes=False),
  )
  def deg_kernel(dst_hbm, out_hbm, dst_v, cnt_v):
    c = lax.axis_index("c")
    s = lax.axis_index("s")
    w = c * NS + s
    zeros = jnp.zeros((LANES,), jnp.float32)

    def zbody(i, carry):
      cnt_v[pl.ds(i * LANES, LANES)] = zeros
      return carry

    lax.fori_loop(0, n_cnt // LANES, zbody, 0)
    pltpu.sync_copy(dst_hbm.at[pl.ds(w * e_per_tile, e_per_tile)], dst_v)
    ones = jnp.ones((LANES,), jnp.float32)

    def body(i, carry):
      idx = dst_v[pl.ds(i * LANES, LANES)]
      plsc.addupdate_scatter(cnt_v, [idx], ones)
      return carry

    lax.fori_loop(0, e_per_tile // LANES, body, 0)
    pltpu.sync_copy(cnt_v, out_hbm.at[w])

  return deg_kernel


def _make_agg(n_pad, dh, e_pad):
  rows_per_tile = n_pad // NS
  chunks_per_tile = e_pad // (NS * KCH)
  nbuf = 4
  n_groups = 2
  cpg = chunks_per_tile // n_groups
  assert chunks_per_tile == n_groups * cpg
  assert cpg % nbuf == 0 and cpg >= 2 * nbuf
  mesh = plsc.VectorSubcoreMesh(core_axis_name="c", subcore_axis_name="s")

  @functools.partial(
      pl.kernel,
      out_type=jax.ShapeDtypeStruct((NC, n_pad, dh), jnp.bfloat16),
      mesh=mesh,
      scratch_types=[
          pltpu.VMEM_SHARED((n_pad, dh), jnp.bfloat16),
          pltpu.VMEM((cpg, 2, KCH), jnp.int32),
          [pltpu.VMEM((KCH, dh), jnp.bfloat16)] * nbuf,
          [pltpu.SemaphoreType.DMA] * nbuf,
          [pltpu.SemaphoreType.DMA] * nbuf,
      ],
      compiler_params=pltpu.CompilerParams(
          needs_layout_passes=False, use_tc_tiling_on_sc=False),
  )
  def agg_kernel(hs_hbm, sd_hbm, out_hbm, acc, idx_sd, rows, sem_g, sem_s):
    c = lax.axis_index("c")
    s = lax.axis_index("s")
    base = s * rows_per_tile

    def gather(b, j):
      return pltpu.make_async_copy(hs_hbm.at[c].at[idx_sd.at[j, 0]], rows[b],
                                   sem_g[b])

    def scatter(b, j):
      return pltpu.make_async_copy(rows[b], acc.at[idx_sd.at[j, 1]], sem_s[b])

    pltpu.sync_copy(hs_hbm.at[c].at[pl.ds(base, rows_per_tile)],
                    acc.at[pl.ds(base, rows_per_tile)])
    plsc.subcore_barrier()

    for g in range(n_groups):
      pltpu.sync_copy(
          sd_hbm.at[pl.ds(s * chunks_per_tile + g * cpg, cpg)], idx_sd)
      for b in range(nbuf):
        gather(b, b).start()

      def body(t, carry):
        for b in range(nbuf):
          j = t * nbuf + b
          gather(b, j).wait()
          scatter(b, j).start(add=True)

          @pl.when(j + nbuf < cpg)
          def _():
            scatter(b, j).wait()
            gather(b, j + nbuf).start()
        return carry

      lax.fori_loop(0, cpg // nbuf, body, 0)
      for b in range(nbuf):
        scatter(b, 0).wait()

    plsc.subcore_barrier()
    pltpu.sync_copy(acc.at[pl.ds(base, rows_per_tile)],
                    out_hbm.at[c].at[pl.ds(base, rows_per_tile)])

  return agg_kernel


def _make_agg_es(n_pad, d, e_pad):
  rows_per_tile = n_pad // NS
  chunks_per_core = e_pad // (NC * KCH)
  cpt = chunks_per_core // NS
  nbuf = 4
  assert cpt % nbuf == 0
  zrows = 128
  assert rows_per_tile % zrows == 0
  mesh = plsc.VectorSubcoreMesh(core_axis_name="c", subcore_axis_name="s")

  @functools.partial(
      pl.kernel,
      out_type=jax.ShapeDtypeStruct((NC, n_pad, d), jnp.float32),
      mesh=mesh,
      scratch_types=[
          pltpu.VMEM_SHARED((n_pad, d), jnp.float32),
          pltpu.VMEM((cpt, 2, KCH), jnp.int32),
          pltpu.VMEM((zrows, d), jnp.float32),
          [pltpu.VMEM((KCH, d), jnp.float32)] * nbuf,
          [pltpu.SemaphoreType.DMA] * nbuf,
          [pltpu.SemaphoreType.DMA] * nbuf,
      ],
      compiler_params=pltpu.CompilerParams(
          needs_layout_passes=False, use_tc_tiling_on_sc=False),
  )
  def agg_kernel(hs_hbm, sd_hbm, out_hbm, acc, idx_sd, zbuf, rows, sem_g,
                 sem_s):
    c = lax.axis_index("c")
    s = lax.axis_index("s")
    base = s * rows_per_tile

    def gather(b, j):
      return pltpu.make_async_copy(hs_hbm.at[idx_sd.at[j, 0]], rows[b],
                                   sem_g[b])

    def scatter(b, j):
      return pltpu.make_async_copy(rows[b], acc.at[idx_sd.at[j, 1]], sem_s[b])

    @pl.when(c == 0)
    def _():
      pltpu.sync_copy(hs_hbm.at[pl.ds(base, rows_per_tile)],
                      acc.at[pl.ds(base, rows_per_tile)])

    @pl.when(c == 1)
    def _():
      zv = jnp.zeros((LANES,), jnp.float32)

      def zb(i, carry):
        for k in range(d // LANES):
          zbuf[i, pl.ds(k * LANES, LANES)] = zv
        return carry

      lax.fori_loop(0, zrows, zb, 0)
      for r in range(rows_per_tile // zrows):
        pltpu.sync_copy(zbuf, acc.at[pl.ds(base + r * zrows, zrows)])

    plsc.subcore_barrier()
    pltpu.sync_copy(
        sd_hbm.at[pl.ds(c * chunks_per_core + s * cpt, cpt)], idx_sd)
    for b in range(nbuf):
      gather(b, b).start()

    def body(t, carry):
      for b in range(nbuf):
        j = t * nbuf + b
        gather(b, j).wait()
        scatter(b, j).start(add=True)

        @pl.when(j + nbuf < cpt)
        def _():
          scatter(b, j).wait()
          gather(b, j + nbuf).start()
      return carry

    lax.fori_loop(0, cpt // nbuf, body, 0)
    for b in range(nbuf):
      scatter(b, 0).wait()
    plsc.subcore_barrier()
    pltpu.sync_copy(acc.at[pl.ds(base, rows_per_tile)],
                    out_hbm.at[c].at[pl.ds(base, rows_per_tile)])

  return agg_kernel


def _dis_from_parts(degp_block):
  deg = jnp.sum(degp_block, axis=0) + 1.0
  return lax.rsqrt(deg)


def _make_mm1(n, n_pad, d_in, d_hid, bm):
  dh = d_hid // NC

  def body(x_ref, w_ref, degp_ref, o_ref):
    dis = _dis_from_parts(degp_ref[...])
    h = jnp.dot(x_ref[...].astype(jnp.bfloat16),
                w_ref[...].astype(jnp.bfloat16),
                preferred_element_type=jnp.float32)
    o_ref[0] = (h * dis[:, None]).astype(jnp.bfloat16)

  return pl.pallas_call(
      body,
      grid=(NC, n_pad // bm),
      in_specs=[
          pl.BlockSpec((bm, d_in), lambda c, i: (i, 0)),
          pl.BlockSpec((d_in, dh), lambda c, i: (0, c)),
          pl.BlockSpec((NC * NS, bm), lambda c, i: (0, i)),
      ],
      out_specs=pl.BlockSpec((1, bm, dh), lambda c, i: (c, i, 0)),
      out_shape=jax.ShapeDtypeStruct((NC, n_pad, dh), jnp.bfloat16),
  )


def _make_mm2(n_pad, d_hid, d_out, bm):
  dh_in = d_hid // NC

  def body(a_ref, degp_ref, b1_ref, w2_ref, o_ref):
    dis = _dis_from_parts(degp_ref[...])
    z = jnp.concatenate([a_ref[0], a_ref[1]], axis=-1).astype(jnp.float32)
    z = jnp.maximum(z * dis[:, None] + b1_ref[...], 0.0)
    h = jnp.dot(z.astype(jnp.bfloat16), w2_ref[...].astype(jnp.bfloat16),
                preferred_element_type=jnp.float32)
    o_ref[...] = h * dis[:, None]

  return pl.pallas_call(
      body,
      grid=(n_pad // bm,),
      in_specs=[
          pl.BlockSpec((NC, bm, dh_in), lambda i: (0, i, 0)),
          pl.BlockSpec((NC * NS, bm), lambda i: (0, i)),
          pl.BlockSpec((1, d_hid), lambda i: (0, 0)),
          pl.BlockSpec((d_hid, d_out), lambda i: (0, 0)),
      ],
      out_specs=pl.BlockSpec((bm, d_out), lambda i: (i, 0)),
      out_shape=jax.ShapeDtypeStruct((n_pad, d_out), jnp.float32),
  )


def _make_fin(n, n_pad, d_out, n_cls, bm):
  def body(a_ref, degp_ref, b2_ref, wc_ref, bc_ref, o_ref):
    dis = _dis_from_parts(degp_ref[...])
    z = a_ref[0] + a_ref[1]
    z = z * dis[:, None] + b2_ref[...]
    o_ref[...] = (
        jnp.dot(z, wc_ref[...], preferred_element_type=jnp.float32)
        + bc_ref[...])

  return pl.pallas_call(
      body,
      grid=(n_pad // bm,),
      in_specs=[
          pl.BlockSpec((NC, bm, d_out), lambda i: (0, i, 0)),
          pl.BlockSpec((NC * NS, bm), lambda i: (0, i)),
          pl.BlockSpec((1, d_out), lambda i: (0, 0)),
          pl.BlockSpec((d_out, n_cls), lambda i: (0, 0)),
          pl.BlockSpec((1, n_cls), lambda i: (0, 0)),
      ],
      out_specs=pl.BlockSpec((bm, n_cls), lambda i: (i, 0)),
      out_shape=jax.ShapeDtypeStruct((n, n_cls), jnp.float32),
  )


def kernel(x, edge_index, W1, b1, W2, b2, Wc, bc):
  n, d_in = x.shape
  d_hid = W1.shape[1]
  d_out = W2.shape[1]
  n_cls = Wc.shape[1]
  e = edge_index.shape[1]

  quantum = 2 * NS * KCH
  e_pad = ((e + quantum - 1) // quantum) * quantum
  pad = e_pad - e
  src = edge_index[0].astype(jnp.int32)
  dst = edge_index[1].astype(jnp.int32)
  bm = 1024
  n_pad = ((n + bm - 1) // bm) * bm
  src_p = jnp.concatenate([src, jnp.arange(pad, dtype=jnp.int32) % n])
  dst_p = jnp.concatenate(
      [dst, n + (jnp.arange(pad, dtype=jnp.int32) % (n_pad - n))])
  sd2 = jnp.stack([src_p.reshape(-1, KCH), dst_p.reshape(-1, KCH)], axis=1)
  degp = _make_deg(n_pad, e_pad)(dst_p)
  hs1 = _make_mm1(n, n_pad, d_in, d_hid, bm)(x, W1, degp)
  acc1 = _make_agg(n_pad, d_hid // NC, e_pad)(hs1, sd2)
  hs2 = _make_mm2(n_pad, d_hid, d_out, bm)(acc1, degp, b1.reshape(1, -1), W2)
  acc2 = _make_agg_es(n_pad, d_out, e_pad)(hs2, sd2)
  logits = _make_fin(n, n_pad, d_out, n_cls, bm)(
      acc2, degp, b2.reshape(1, -1), Wc, bc.reshape(1, -1))
  return logits
